# baseline (speedup 1.0000x reference)
import functools

import jax
import jax.numpy as jnp
import jax.scipy.linalg as jsl
from jax import lax
from jax.experimental import pallas as pl
from jax.experimental.pallas import tpu as pltpu
from jax.experimental.pallas import tpu_sc as plsc

N_NODES = 100000
N_EDGES = 3200000
LATENT = 16
EP = N_EDGES // 8

NC = 2
NS = 16
NW = NC * NS
EW = N_EDGES // NW
CHUNK = 800
NCH = EW // CHUNK
HALF = N_NODES // NC
STRIPE2 = HALF // NS
EW2 = N_EDGES // NS
NCH2 = EW2 // CHUNK

F32 = jnp.float32


def _leaky(x):
    return jnp.where(x >= 0, x, 0.01 * x)


def _bd(W, k):
    return jsl.block_diag(*([W] * k))


def _tc_mlp(xs, Ws, bi, WhT, bh, WoT, bo, gb, residual, block_rows):
    nx = len(xs)
    rows = xs[0].shape[0]
    out_f = WoT.shape[1]
    has_ln = gb is not None
    grid = rows // block_rows

    def body(*refs):
        xrefs = refs[:nx]
        wrefs = refs[nx:2 * nx]
        k = 2 * nx
        bi_r, wh_r, bh_r, wo_r, bo_r = refs[k:k + 5]
        k += 5
        if has_ln:
            g_r, b_r = refs[k:k + 2]
            k += 2
        out_ref = refs[k]

        acc = None
        for xr, wr in zip(xrefs, wrefs):
            t = jnp.dot(xr[...], wr[...], preferred_element_type=F32)
            acc = t if acc is None else acc + t
        f = _leaky(acc + bi_r[...])
        h = _leaky(jnp.dot(f, wh_r[...], preferred_element_type=F32) + bh_r[...])
        o = jnp.dot(h, wo_r[...], preferred_element_type=F32) + bo_r[...]
        if has_ln:
            mu = jnp.mean(o, axis=-1, keepdims=True)
            var = jnp.mean((o - mu) ** 2, axis=-1, keepdims=True)
            o = (o - mu) * lax.rsqrt(var + 1e-5) * g_r[...] + b_r[...]
        if residual:
            o = o + xrefs[0][...]
        out_ref[...] = o

    in_specs = [pl.BlockSpec((block_rows, x.shape[1]), lambda i: (i, 0))
                for x in xs]
    for w in list(Ws) + [bi, WhT, bh, WoT, bo] + (list(gb) if has_ln else []):
        in_specs.append(pl.BlockSpec(w.shape, lambda i: (0, 0)))
    operands = list(xs) + list(Ws) + [bi, WhT, bh, WoT, bo]
    if has_ln:
        operands += list(gb)

    return pl.pallas_call(
        body,
        grid=(grid,),
        in_specs=in_specs,
        out_specs=pl.BlockSpec((block_rows, out_f), lambda i: (i, 0)),
        out_shape=jax.ShapeDtypeStruct((rows, out_f), F32),
    )(*operands)


def _prep(p):
    WiT = p['Wi'].T
    bi = p['bi'][None, :]
    WhT = p['hidden'][0][0].T
    bh = p['hidden'][0][1][None, :]
    WoT = p['Wo'].T
    bo = p['bo'][None, :]
    gb = (p['g'][None, :], p['b'][None, :]) if 'g' in p else None
    return WiT, bi, WhT, bh, WoT, bo, gb


def _prep_packed8(p, in_splits):
    WiT = p['Wi'].T
    Ws8 = []
    o = 0
    for w in in_splits:
        Ws8.append(_bd(WiT[o:o + w], 8))
        o += w
    bi8 = jnp.tile(p['bi'][None, :], (1, 8))
    Wh8 = _bd(p['hidden'][0][0].T, 8)
    bh8 = jnp.tile(p['hidden'][0][1][None, :], (1, 8))
    Wo8 = _bd(p['Wo'].T, 8)
    bo8 = jnp.tile(p['bo'][None, :], (1, 8))
    A8 = _bd(jnp.full((LATENT, LATENT), 1.0 / LATENT, F32), 8)
    g8 = jnp.tile(p['g'][None, :], (1, 8))
    b8 = jnp.tile(p['b'][None, :], (1, 8))
    return Ws8, bi8, Wh8, bh8, Wo8, bo8, A8, g8, b8


def _packed_tail(h, Wo8, bo8, A8, g8, b8, res):
    o = jnp.dot(h, Wo8, preferred_element_type=F32) + bo8
    mu = jnp.dot(o, A8, preferred_element_type=F32)
    d = o - mu
    var = jnp.dot(d * d, A8, preferred_element_type=F32)
    o = d * lax.rsqrt(var + 1e-5) * g8 + b8
    return o if res is None else o + res


def _tc_proc_e(xs_p, pw, block_rows=4000):
    Ws8, bi8, Wh8, bh8, Wo8, bo8, A8, g8, b8 = pw
    nx = len(xs_p)

    def body(*refs):
        xrefs = refs[:nx]
        wrefs = refs[nx:2 * nx]
        bi_r, wh_r, bh_r, wo_r, bo_r, a_r, g_r, b_r = refs[2 * nx:2 * nx + 8]
        out_ref = refs[2 * nx + 8]
        acc = None
        for xr, wr in zip(xrefs, wrefs):
            t = jnp.dot(xr[...], wr[...], preferred_element_type=F32)
            acc = t if acc is None else acc + t
        f = _leaky(acc + bi_r[...])
        h = _leaky(jnp.dot(f, wh_r[...], preferred_element_type=F32) + bh_r[...])
        out_ref[...] = _packed_tail(h, wo_r[...], bo_r[...], a_r[...],
                                    g_r[...], b_r[...], xrefs[0][...])

    in_specs = [pl.BlockSpec((block_rows, 128), lambda i: (i, 0))
                for _ in xs_p]
    consts = list(Ws8) + [bi8, Wh8, bh8, Wo8, bo8, A8, g8, b8]
    for w in consts:
        in_specs.append(pl.BlockSpec(w.shape, lambda i: (0, 0)))

    rows = xs_p[0].shape[0]
    return pl.pallas_call(
        body,
        grid=(rows // block_rows,),
        in_specs=in_specs,
        out_specs=pl.BlockSpec((block_rows, 128), lambda i: (i, 0)),
        out_shape=jax.ShapeDtypeStruct((rows, 128), F32),
    )(*(list(xs_p) + consts))


def _tc_enc_e(efp, p, block_rows=1000):
    W32 = _bd(p['Wi'].T, 32)
    bi32 = jnp.tile(p['bi'][None, :], (1, 32))
    _, _, Wh8, bh8, Wo8, bo8, A8, g8, b8 = _prep_packed8(p, [])

    def body(x_ref, w1, b1, wh, bhh, wo, bob, a_r, g_r, b_r, out_ref):
        x = x_ref[...]
        f = _leaky(jnp.dot(x, w1[...], preferred_element_type=F32) + b1[...])
        f8 = jnp.reshape(f, (4 * block_rows, 256))
        h = _leaky(jnp.dot(f8, wh[...], preferred_element_type=F32) + bhh[...])
        out_ref[...] = _packed_tail(h, wo[...], bob[...], a_r[...],
                                    g_r[...], b_r[...], None)

    consts = [W32, bi32, Wh8, bh8, Wo8, bo8, A8, g8, b8]
    in_specs = [pl.BlockSpec((block_rows, 128), lambda i: (i, 0))]
    for w in consts:
        in_specs.append(pl.BlockSpec(w.shape, lambda i: (0, 0)))

    return pl.pallas_call(
        body,
        grid=(efp.shape[0] // block_rows,),
        in_specs=in_specs,
        out_specs=pl.BlockSpec((4 * block_rows, 128), lambda i: (i, 0)),
        out_shape=jax.ShapeDtypeStruct((4 * efp.shape[0], 128), F32),
    )(*([efp] + consts))


def _sc_gather(n, src, dst):
    mesh = plsc.VectorSubcoreMesh(core_axis_name="c", subcore_axis_name="s")

    @functools.partial(
        pl.kernel,
        out_type=(jax.ShapeDtypeStruct((N_EDGES, LATENT), F32),
                  jax.ShapeDtypeStruct((N_EDGES, LATENT), F32)),
        mesh=mesh,
        scratch_types=[
            pltpu.VMEM((CHUNK,), jnp.int32),
            pltpu.VMEM((CHUNK,), jnp.int32),
            pltpu.VMEM((CHUNK, LATENT), F32),
            pltpu.VMEM((CHUNK, LATENT), F32),
            pltpu.SemaphoreType.DMA,
            pltpu.SemaphoreType.DMA,
        ],
        compiler_params=pltpu.CompilerParams(use_tc_tiling_on_sc=False),
    )
    def gather_k(n_hbm, src_hbm, dst_hbm, gs_hbm, gd_hbm,
                 is_v, id_v, rs_v, rd_v, sem_s, sem_d):
        wid = lax.axis_index("s") * NC + lax.axis_index("c")
        base = wid * EW

        def step(i, carry):
            off = base + i * CHUNK
            pltpu.sync_copy(src_hbm.at[pl.ds(off, CHUNK)], is_v)
            pltpu.sync_copy(dst_hbm.at[pl.ds(off, CHUNK)], id_v)
            d1 = pltpu.async_copy(n_hbm.at[is_v], rs_v, sem_s)
            d2 = pltpu.async_copy(n_hbm.at[id_v], rd_v, sem_d)
            d1.wait()
            d2.wait()
            pltpu.sync_copy(rs_v, gs_hbm.at[pl.ds(off, CHUNK)])
            pltpu.sync_copy(rd_v, gd_hbm.at[pl.ds(off, CHUNK)])
            return carry

        lax.fori_loop(0, NCH, step, 0)

    return gather_k(n, src, dst)


def _sc_scatter(e, dst):
    mesh = plsc.VectorSubcoreMesh(core_axis_name="c", subcore_axis_name="s")

    @functools.partial(
        pl.kernel,
        out_type=jax.ShapeDtypeStruct((N_NODES, LATENT), F32),
        mesh=mesh,
        scratch_types=[
            pltpu.VMEM((CHUNK,), jnp.int32),
            pltpu.VMEM((CHUNK,), jnp.int32),
            pltpu.VMEM((CHUNK, LATENT), F32),
            pltpu.VMEM((STRIPE2, LATENT), F32),
            pltpu.VMEM_SHARED((HALF + 8, LATENT), F32),
        ],
        compiler_params=pltpu.CompilerParams(use_tc_tiling_on_sc=False),
    )
    def scatter_k(e_hbm, dst_hbm, out_hbm, raw_v, idx_v, rows_v, zbuf_v, acc_sh):
        c = lax.axis_index("c")
        s = lax.axis_index("s")
        lo = c * HALF

        def zstep(i, carry):
            zbuf_v[i, :] = jnp.zeros((LATENT,), F32)
            return carry

        lax.fori_loop(0, STRIPE2, zstep, 0)
        pltpu.sync_copy(zbuf_v, acc_sh.at[pl.ds(s * STRIPE2, STRIPE2)])
        plsc.subcore_barrier()

        base = s * EW2

        def step(i, carry):
            off = base + i * CHUNK
            pltpu.sync_copy(dst_hbm.at[pl.ds(off, CHUNK)], raw_v)
            pltpu.sync_copy(e_hbm.at[pl.ds(off, CHUNK)], rows_v)
            for j in range(CHUNK // 16):
                v = raw_v[pl.ds(j * 16, 16)] - lo
                ok = (v >= 0) & (v < HALF)
                idx_v[pl.ds(j * 16, 16)] = jnp.where(ok, v, HALF)
            pltpu.sync_copy(rows_v, acc_sh.at[idx_v], add=True)
            return carry

        lax.fori_loop(0, NCH2, step, 0)
        plsc.subcore_barrier()

        pltpu.sync_copy(acc_sh.at[pl.ds(s * STRIPE2, STRIPE2)], zbuf_v)
        pltpu.sync_copy(zbuf_v, out_hbm.at[pl.ds(lo + s * STRIPE2, STRIPE2)])

    return scatter_k(e, dst)


def kernel(nfeatures, efeatures, params, edge_index):
    src = edge_index[0]
    dst = edge_index[1]

    WiT, bi, WhT, bh, WoT, bo, gb = _prep(params['enc_n'])
    n = _tc_mlp([nfeatures], [WiT], bi, WhT, bh, WoT, bo, gb,
                residual=False, block_rows=10000)

    efp = efeatures.reshape(N_EDGES // 32, 128)
    e_p = _tc_enc_e(efp, params['enc_e'])

    for it in range(2):
        gs, gd = _sc_gather(n, src, dst)
        pw = _prep_packed8(params['proc_e'][it], [16, 16, 16])
        e_p = _tc_proc_e([e_p, gs.reshape(EP, 128), gd.reshape(EP, 128)], pw)

        pe = _sc_scatter(e_p.reshape(N_EDGES, LATENT), dst)
        WiT, bi, WhT, bh, WoT, bo, gb = _prep(params['proc_n'][it])
        n = _tc_mlp([n, pe], [WiT[0:16], WiT[16:32]],
                    bi, WhT, bh, WoT, bo, gb,
                    residual=True, block_rows=10000)

    WiT, bi, WhT, bh, WoT, bo, gb = _prep(params['dec'])
    return _tc_mlp([n], [WiT], bi, WhT, bh, WoT, bo, gb,
                   residual=False, block_rows=10000)


# speedup vs baseline: 1.6181x; 1.6181x over previous
import functools

import jax
import jax.numpy as jnp
import jax.scipy.linalg as jsl
from jax import lax
from jax.experimental import pallas as pl
from jax.experimental.pallas import tpu as pltpu
from jax.experimental.pallas import tpu_sc as plsc

N_NODES = 100000
N_EDGES = 3200000
LATENT = 16
EP = N_EDGES // 8

NC = 2
NS = 16
NW = NC * NS
EW = N_EDGES // NW
CHUNK = 800
NCH = EW // CHUNK
HALF = N_NODES // NC
STRIPE2 = HALF // NS
EW2 = N_EDGES // NS
NCH2 = EW2 // CHUNK

F32 = jnp.float32


def _leaky(x):
    return jnp.where(x >= 0, x, 0.01 * x)


def _bd(W, k):
    return jsl.block_diag(*([W] * k))


def _tc_mlp(xs, Ws, bi, WhT, bh, WoT, bo, gb, residual, block_rows):
    nx = len(xs)
    rows = xs[0].shape[0]
    out_f = WoT.shape[1]
    has_ln = gb is not None
    grid = rows // block_rows

    def body(*refs):
        xrefs = refs[:nx]
        wrefs = refs[nx:2 * nx]
        k = 2 * nx
        bi_r, wh_r, bh_r, wo_r, bo_r = refs[k:k + 5]
        k += 5
        if has_ln:
            g_r, b_r = refs[k:k + 2]
            k += 2
        out_ref = refs[k]

        acc = None
        for xr, wr in zip(xrefs, wrefs):
            t = jnp.dot(xr[...], wr[...], preferred_element_type=F32)
            acc = t if acc is None else acc + t
        f = _leaky(acc + bi_r[...])
        h = _leaky(jnp.dot(f, wh_r[...], preferred_element_type=F32) + bh_r[...])
        o = jnp.dot(h, wo_r[...], preferred_element_type=F32) + bo_r[...]
        if has_ln:
            mu = jnp.mean(o, axis=-1, keepdims=True)
            var = jnp.mean((o - mu) ** 2, axis=-1, keepdims=True)
            o = (o - mu) * lax.rsqrt(var + 1e-5) * g_r[...] + b_r[...]
        if residual:
            o = o + xrefs[0][...]
        out_ref[...] = o

    in_specs = [pl.BlockSpec((block_rows, x.shape[1]), lambda i: (i, 0))
                for x in xs]
    for w in list(Ws) + [bi, WhT, bh, WoT, bo] + (list(gb) if has_ln else []):
        in_specs.append(pl.BlockSpec(w.shape, lambda i: (0, 0)))
    operands = list(xs) + list(Ws) + [bi, WhT, bh, WoT, bo]
    if has_ln:
        operands += list(gb)

    return pl.pallas_call(
        body,
        grid=(grid,),
        in_specs=in_specs,
        out_specs=pl.BlockSpec((block_rows, out_f), lambda i: (i, 0)),
        out_shape=jax.ShapeDtypeStruct((rows, out_f), F32),
    )(*operands)


def _prep(p):
    WiT = p['Wi'].T
    bi = p['bi'][None, :]
    WhT = p['hidden'][0][0].T
    bh = p['hidden'][0][1][None, :]
    WoT = p['Wo'].T
    bo = p['bo'][None, :]
    gb = (p['g'][None, :], p['b'][None, :]) if 'g' in p else None
    return WiT, bi, WhT, bh, WoT, bo, gb


def _prep_packed8(p, in_splits):
    WiT = p['Wi'].T
    Ws8 = []
    o = 0
    for w in in_splits:
        Ws8.append(_bd(WiT[o:o + w], 8))
        o += w
    bi8 = jnp.tile(p['bi'][None, :], (1, 8))
    Wh8 = _bd(p['hidden'][0][0].T, 8)
    bh8 = jnp.tile(p['hidden'][0][1][None, :], (1, 8))
    Wo8 = _bd(p['Wo'].T, 8)
    bo8 = jnp.tile(p['bo'][None, :], (1, 8))
    A8 = _bd(jnp.full((LATENT, LATENT), 1.0 / LATENT, F32), 8)
    g8 = jnp.tile(p['g'][None, :], (1, 8))
    b8 = jnp.tile(p['b'][None, :], (1, 8))
    return Ws8, bi8, Wh8, bh8, Wo8, bo8, A8, g8, b8


def _packed_tail(h, Wo8, bo8, A8, g8, b8, res):
    o = jnp.dot(h, Wo8, preferred_element_type=F32) + bo8
    mu = jnp.dot(o, A8, preferred_element_type=F32)
    d = o - mu
    var = jnp.dot(d * d, A8, preferred_element_type=F32)
    o = d * lax.rsqrt(var + 1e-5) * g8 + b8
    return o if res is None else o + res


def _tc_proc_e(xs_p, pw, block_rows=4000):
    Ws8, bi8, Wh8, bh8, Wo8, bo8, A8, g8, b8 = pw
    nx = len(xs_p)

    def body(*refs):
        xrefs = refs[:nx]
        wrefs = refs[nx:2 * nx]
        bi_r, wh_r, bh_r, wo_r, bo_r, a_r, g_r, b_r = refs[2 * nx:2 * nx + 8]
        out_ref = refs[2 * nx + 8]
        acc = None
        for xr, wr in zip(xrefs, wrefs):
            t = jnp.dot(xr[...], wr[...], preferred_element_type=F32)
            acc = t if acc is None else acc + t
        f = _leaky(acc + bi_r[...])
        h = _leaky(jnp.dot(f, wh_r[...], preferred_element_type=F32) + bh_r[...])
        out_ref[...] = _packed_tail(h, wo_r[...], bo_r[...], a_r[...],
                                    g_r[...], b_r[...], xrefs[0][...])

    in_specs = [pl.BlockSpec((block_rows, 128), lambda i: (i, 0))
                for _ in xs_p]
    consts = list(Ws8) + [bi8, Wh8, bh8, Wo8, bo8, A8, g8, b8]
    for w in consts:
        in_specs.append(pl.BlockSpec(w.shape, lambda i: (0, 0)))

    rows = xs_p[0].shape[0]
    return pl.pallas_call(
        body,
        grid=(rows // block_rows,),
        in_specs=in_specs,
        out_specs=pl.BlockSpec((block_rows, 128), lambda i: (i, 0)),
        out_shape=jax.ShapeDtypeStruct((rows, 128), F32),
    )(*(list(xs_p) + consts))


def _tc_enc_e(cols, p, b0=200):
    rows_c = cols[0].shape[0]
    W32 = _bd(p['Wi'].T, 32)
    bi32 = jnp.tile(p['bi'][None, :], (1, 32))
    _, _, Wh8, bh8, Wo8, bo8, A8, g8, b8 = _prep_packed8(p, [])
    eye4 = jnp.eye(4, dtype=F32)
    Ps = [jnp.kron(jnp.eye(128, dtype=F32), eye4[k][None, :]) for k in range(4)]

    def body(c0, c1, c2, c3, p0, p1, p2, p3,
             w1, b1, wh, bhh, wo, bob, a_r, g_r, b_r, out_ref):
        acc = None
        for cr, pr in zip((c0, c1, c2, c3), (p0, p1, p2, p3)):
            t = jnp.dot(cr[...], pr[...], preferred_element_type=F32)
            acc = t if acc is None else acc + t
        x = jnp.reshape(acc, (4 * b0, 128))
        f = _leaky(jnp.dot(x, w1[...], preferred_element_type=F32) + b1[...])
        f8 = jnp.reshape(f, (16 * b0, 256))
        h = _leaky(jnp.dot(f8, wh[...], preferred_element_type=F32) + bhh[...])
        out_ref[...] = _packed_tail(h, wo[...], bob[...], a_r[...],
                                    g_r[...], b_r[...], None)

    consts = Ps + [W32, bi32, Wh8, bh8, Wo8, bo8, A8, g8, b8]
    in_specs = [pl.BlockSpec((b0, 128), lambda i: (i, 0)) for _ in range(4)]
    for w in consts:
        in_specs.append(pl.BlockSpec(w.shape, lambda i: (0, 0)))

    return pl.pallas_call(
        body,
        grid=(rows_c // b0,),
        in_specs=in_specs,
        out_specs=pl.BlockSpec((16 * b0, 128), lambda i: (i, 0)),
        out_shape=jax.ShapeDtypeStruct((16 * rows_c, 128), F32),
    )(*(list(cols) + consts))


def _sc_gather(n, src, dst):
    mesh = plsc.VectorSubcoreMesh(core_axis_name="c", subcore_axis_name="s")

    @functools.partial(
        pl.kernel,
        out_type=(jax.ShapeDtypeStruct((N_EDGES, LATENT), F32),
                  jax.ShapeDtypeStruct((N_EDGES, LATENT), F32)),
        mesh=mesh,
        scratch_types=[
            pltpu.VMEM((CHUNK,), jnp.int32),
            pltpu.VMEM((CHUNK,), jnp.int32),
            pltpu.VMEM((CHUNK, LATENT), F32),
            pltpu.VMEM((CHUNK, LATENT), F32),
            pltpu.SemaphoreType.DMA,
            pltpu.SemaphoreType.DMA,
        ],
        compiler_params=pltpu.CompilerParams(use_tc_tiling_on_sc=False),
    )
    def gather_k(n_hbm, src_hbm, dst_hbm, gs_hbm, gd_hbm,
                 is_v, id_v, rs_v, rd_v, sem_s, sem_d):
        wid = lax.axis_index("s") * NC + lax.axis_index("c")
        base = wid * EW

        def step(i, carry):
            off = base + i * CHUNK
            pltpu.sync_copy(src_hbm.at[pl.ds(off, CHUNK)], is_v)
            pltpu.sync_copy(dst_hbm.at[pl.ds(off, CHUNK)], id_v)
            d1 = pltpu.async_copy(n_hbm.at[is_v], rs_v, sem_s)
            d2 = pltpu.async_copy(n_hbm.at[id_v], rd_v, sem_d)
            d1.wait()
            d2.wait()
            pltpu.sync_copy(rs_v, gs_hbm.at[pl.ds(off, CHUNK)])
            pltpu.sync_copy(rd_v, gd_hbm.at[pl.ds(off, CHUNK)])
            return carry

        lax.fori_loop(0, NCH, step, 0)

    return gather_k(n, src, dst)


def _sc_scatter(e, dst):
    mesh = plsc.VectorSubcoreMesh(core_axis_name="c", subcore_axis_name="s")

    @functools.partial(
        pl.kernel,
        out_type=jax.ShapeDtypeStruct((N_NODES, LATENT), F32),
        mesh=mesh,
        scratch_types=[
            pltpu.VMEM((CHUNK,), jnp.int32),
            pltpu.VMEM((CHUNK,), jnp.int32),
            pltpu.VMEM((CHUNK, LATENT), F32),
            pltpu.VMEM((STRIPE2, LATENT), F32),
            pltpu.VMEM_SHARED((HALF + 8, LATENT), F32),
        ],
        compiler_params=pltpu.CompilerParams(use_tc_tiling_on_sc=False),
    )
    def scatter_k(e_hbm, dst_hbm, out_hbm, raw_v, idx_v, rows_v, zbuf_v, acc_sh):
        c = lax.axis_index("c")
        s = lax.axis_index("s")
        lo = c * HALF

        def zstep(i, carry):
            zbuf_v[i, :] = jnp.zeros((LATENT,), F32)
            return carry

        lax.fori_loop(0, STRIPE2, zstep, 0)
        pltpu.sync_copy(zbuf_v, acc_sh.at[pl.ds(s * STRIPE2, STRIPE2)])
        plsc.subcore_barrier()

        base = s * EW2

        def step(i, carry):
            off = base + i * CHUNK
            pltpu.sync_copy(dst_hbm.at[pl.ds(off, CHUNK)], raw_v)
            pltpu.sync_copy(e_hbm.at[pl.ds(off, CHUNK)], rows_v)
            for j in range(CHUNK // 16):
                v = raw_v[pl.ds(j * 16, 16)] - lo
                ok = (v >= 0) & (v < HALF)
                idx_v[pl.ds(j * 16, 16)] = jnp.where(ok, v, HALF)
            pltpu.sync_copy(rows_v, acc_sh.at[idx_v], add=True)
            return carry

        lax.fori_loop(0, NCH2, step, 0)
        plsc.subcore_barrier()

        pltpu.sync_copy(acc_sh.at[pl.ds(s * STRIPE2, STRIPE2)], zbuf_v)
        pltpu.sync_copy(zbuf_v, out_hbm.at[pl.ds(lo + s * STRIPE2, STRIPE2)])

    return scatter_k(e, dst)


def kernel(nfeatures, efeatures, params, edge_index):
    src = edge_index[0]
    dst = edge_index[1]

    WiT, bi, WhT, bh, WoT, bo, gb = _prep(params['enc_n'])
    n = _tc_mlp([nfeatures], [WiT], bi, WhT, bh, WoT, bo, gb,
                residual=False, block_rows=10000)

    cols = [efeatures[:, k].reshape(N_EDGES // 128, 128) for k in range(4)]
    e_p = _tc_enc_e(cols, params['enc_e'])

    for it in range(2):
        gs, gd = _sc_gather(n, src, dst)
        pw = _prep_packed8(params['proc_e'][it], [16, 16, 16])
        e_p = _tc_proc_e([e_p, gs.reshape(EP, 128), gd.reshape(EP, 128)], pw)

        pe = _sc_scatter(e_p.reshape(N_EDGES, LATENT), dst)
        WiT, bi, WhT, bh, WoT, bo, gb = _prep(params['proc_n'][it])
        n = _tc_mlp([n, pe], [WiT[0:16], WiT[16:32]],
                    bi, WhT, bh, WoT, bo, gb,
                    residual=True, block_rows=10000)

    WiT, bi, WhT, bh, WoT, bo, gb = _prep(params['dec'])
    return _tc_mlp([n], [WiT], bi, WhT, bh, WoT, bo, gb,
                   residual=False, block_rows=10000)
